# Initial kernel scaffold; baseline (speedup 1.0000x reference)
import jax
import jax.numpy as jnp
from jax.experimental import pallas as pl


def kernel(x, edge_index, W1, b1, W2, b2):
    raise NotImplementedError("write your pallas kernel here")


# speedup vs baseline: 8.3373x; 8.3373x over previous
import dataclasses

import jax
import jax.numpy as jnp
from jax import lax
from jax.experimental import pallas as pl
from jax.experimental.pallas import tpu as pltpu
from jax.experimental.pallas import tpu_sc as plsc

_N = 10000
_E = 320000
_F = 128
_R = 10240
_T = 80
_NC = 2
_NS = 16
_NW = _NC * _NS
_EPW = _E // _NW
_KB = 80
_CHB = _EPW // _KB
_KA = 400
_CHA = _EPW // _KA
_RPS = _R // _NS

_mesh = plsc.VectorSubcoreMesh(core_axis_name="c", subcore_axis_name="s")
_cp = pltpu.CompilerParams()
if "needs_layout_passes" in pltpu.CompilerParams.__dataclass_fields__:
    _cp = dataclasses.replace(_cp, needs_layout_passes=False)


def _f32(*shape):
    return jax.ShapeDtypeStruct(shape, jnp.float32)


def _deg_body(src_hbm, dst_hbm, od_hbm, id_hbm, sidx, didx, od_acc, id_acc):
    cid = lax.axis_index("c")
    sid = lax.axis_index("s")
    wid = sid * _NC + cid

    @pl.loop(0, _R // 16)
    def _(i):
        od_acc.at[pl.ds(i * 16, 16)][...] = jnp.zeros((16,), jnp.float32)
        id_acc.at[pl.ds(i * 16, 16)][...] = jnp.zeros((16,), jnp.float32)

    base = wid * _EPW
    ones = jnp.ones((16,), jnp.float32)

    @pl.loop(0, _CHA)
    def _(j):
        off = base + j * _KA
        pltpu.sync_copy(src_hbm.at[pl.ds(off, _KA)], sidx)
        pltpu.sync_copy(dst_hbm.at[pl.ds(off, _KA)], didx)

        @pl.loop(0, _KA // 16)
        def _(i):
            s16 = sidx[pl.ds(i * 16, 16)]
            d16 = didx[pl.ds(i * 16, 16)]
            plsc.addupdate_scatter(od_acc, [s16], ones)
            plsc.addupdate_scatter(id_acc, [d16], ones)

    pltpu.sync_copy(od_acc, od_hbm.at[wid])
    pltpu.sync_copy(id_acc, id_hbm.at[wid])


@jax.jit
def _deg_call(src, dst):
    k = pl.kernel(
        _deg_body,
        out_type=[_f32(_NW, _R), _f32(_NW, _R)],
        mesh=_mesh,
        compiler_params=_cp,
        scratch_types=[
            pltpu.VMEM((_KA,), jnp.int32),
            pltpu.VMEM((_KA,), jnp.int32),
            pltpu.VMEM((_R,), jnp.float32),
            pltpu.VMEM((_R,), jnp.float32),
        ],
    )
    return k(src, dst)


def _spmm_body(src_hbm, dst_hbm, hs_hbm, nd_hbm, agg_hbm, s_hbm,
               sidx, didx, rows, zrow, zidx, ndv, sacc, agg_sh):
    cid = lax.axis_index("c")
    sid = lax.axis_index("s")
    wid = sid * _NC + cid

    @pl.loop(0, 16)
    def _(i):
        @pl.loop(0, _F, step=16)
        def _(c0):
            zrow.at[i, pl.ds(c0, 16)][...] = jnp.zeros((16,), jnp.float32)

    @pl.loop(0, _R // 16)
    def _(i):
        sacc.at[pl.ds(i * 16, 16)][...] = jnp.zeros((16,), jnp.float32)

    pltpu.sync_copy(nd_hbm, ndv)
    base_row = sid * _RPS

    @pl.loop(0, _RPS // 16)
    def _(b):
        zidx[...] = base_row + b * 16 + lax.iota(jnp.int32, 16)
        pltpu.sync_copy(zrow, agg_sh.at[zidx])

    plsc.subcore_barrier()

    base = wid * _EPW

    @pl.loop(0, _CHB)
    def _(j):
        off = base + j * _KB
        pltpu.sync_copy(src_hbm.at[pl.ds(off, _KB)], sidx)
        pltpu.sync_copy(dst_hbm.at[pl.ds(off, _KB)], didx)
        pltpu.sync_copy(hs_hbm.at[sidx], rows)
        pltpu.sync_copy(rows, agg_sh.at[didx], add=True)

        @pl.loop(0, _KB // 16)
        def _(i):
            d16 = didx[pl.ds(i * 16, 16)]
            s16 = sidx[pl.ds(i * 16, 16)]
            v = plsc.load_gather(ndv, [d16])
            plsc.addupdate_scatter(sacc, [s16], v)

    plsc.subcore_barrier()
    sl = pl.ds(base_row, _RPS)
    pltpu.sync_copy(agg_sh.at[sl], agg_hbm.at[cid, sl])
    pltpu.sync_copy(sacc, s_hbm.at[wid])


@jax.jit
def _spmm_call(src, dst, hs, nd):
    k = pl.kernel(
        _spmm_body,
        out_type=[_f32(_NC, _R, _F), _f32(_NW, _R)],
        mesh=_mesh,
        compiler_params=_cp,
        scratch_types=[
            pltpu.VMEM((_KB,), jnp.int32),
            pltpu.VMEM((_KB,), jnp.int32),
            pltpu.VMEM((_KB, _F), jnp.float32),
            pltpu.VMEM((16, _F), jnp.float32),
            pltpu.VMEM((16,), jnp.int32),
            pltpu.VMEM((_R,), jnp.float32),
            pltpu.VMEM((_R,), jnp.float32),
            pltpu.VMEM_SHARED((_R, _F), jnp.float32),
        ],
    )
    return k(src, dst, hs, nd)


def _prep_body(x_ref, odp_ref, idp_ref, hs_ref, nd_ref, ns_ref):
    od = jnp.sum(odp_ref[...], axis=0)
    ns = lax.rsqrt(jnp.maximum(od, 1.0))
    ind = jnp.sum(idp_ref[...], axis=0)
    nd = lax.rsqrt(jnp.maximum(ind, 1.0))
    hs_ref[...] = x_ref[...] * ns[:, None]
    nd_ref[...] = nd
    ns_ref[...] = ns


def _prep_call(xpad, odp, idp):
    return pl.pallas_call(
        _prep_body,
        grid=(_T,),
        in_specs=[
            pl.BlockSpec((128, _F), lambda i: (i, 0)),
            pl.BlockSpec((_NW, 128), lambda i: (0, i)),
            pl.BlockSpec((_NW, 128), lambda i: (0, i)),
        ],
        out_specs=[
            pl.BlockSpec((128, _F), lambda i: (i, 0)),
            pl.BlockSpec((128,), lambda i: (i,)),
            pl.BlockSpec((128,), lambda i: (i,)),
        ],
        out_shape=[_f32(_R, _F), _f32(_R), _f32(_R)],
    )(xpad, odp, idp)


def _final_body(aggp_ref, nd_ref, ns_ref, sp_ref, w1_ref, b1_ref,
                w2_ref, b2_ref, out_ref, acc_ref):
    i = pl.program_id(0)

    @pl.when(i == 0)
    def _():
        acc_ref[...] = jnp.zeros_like(acc_ref)

    a = (aggp_ref[0] + aggp_ref[1]) * nd_ref[...][:, None]
    h = jnp.dot(a, w1_ref[...], preferred_element_type=jnp.float32) + b1_ref[...]
    r = jnp.maximum(h, 0.0)
    c = ns_ref[...] * jnp.sum(sp_ref[...], axis=0)
    acc_ref[...] += jnp.sum(r * c[:, None], axis=0, keepdims=True)

    @pl.when(i == _T - 1)
    def _():
        v = acc_ref[...] * (1.0 / _N)
        out_ref[...] = (
            jnp.dot(v, w2_ref[...], preferred_element_type=jnp.float32)
            + b2_ref[...]
        )


def _final_call(aggp, nd, ns, sp, W1, b1, W2, b2):
    return pl.pallas_call(
        _final_body,
        grid=(_T,),
        in_specs=[
            pl.BlockSpec((_NC, 128, _F), lambda i: (0, i, 0)),
            pl.BlockSpec((128,), lambda i: (i,)),
            pl.BlockSpec((128,), lambda i: (i,)),
            pl.BlockSpec((_NW, 128), lambda i: (0, i)),
            pl.BlockSpec((_F, _F), lambda i: (0, 0)),
            pl.BlockSpec((1, _F), lambda i: (0, 0)),
            pl.BlockSpec((_F, 16), lambda i: (0, 0)),
            pl.BlockSpec((1, 16), lambda i: (0, 0)),
        ],
        out_specs=pl.BlockSpec((1, 16), lambda i: (0, 0)),
        out_shape=_f32(1, 16),
        scratch_shapes=[pltpu.VMEM((1, _F), jnp.float32)],
    )(aggp, nd, ns, sp, W1, b1, W2, b2)


def kernel(x, edge_index, W1, b1, W2, b2):
    src = edge_index[0].astype(jnp.int32)
    dst = edge_index[1].astype(jnp.int32)
    xpad = jnp.pad(x, ((0, _R - _N), (0, 0)))
    odp, idp = _deg_call(src, dst)
    hs, nd, ns = _prep_call(xpad, odp, idp)
    aggp, sp = _spmm_call(src, dst, hs, nd)
    return _final_call(aggp, nd, ns, sp, W1,
                       b1.reshape(1, _F), W2, b2.reshape(1, 16))
